# baseline (speedup 1.0000x reference)
import functools

import jax
import jax.numpy as jnp
from jax import lax
from jax.experimental import pallas as pl
from jax.experimental.pallas import tpu as pltpu
from jax.experimental.pallas import tpu_sc as plsc

_NC = 2
_NS = 16
_NW = _NC * _NS

_TH = 8
_TW = 128
_PX = _TH * _TW
_G = 8
_CS = 7


def _suppix_unpool_sc(pooled2, spx5, B, C, K, H, W):
    NI = H // _TH
    NJ = W // _TW
    n_blocks = NI * NJ
    n_cq = _NW // B // 2
    cq_sz = C // n_cq
    cv = cq_sz - _CS
    blocks_per_tile = n_blocks // 2

    mesh = plsc.VectorSubcoreMesh(core_axis_name="c", subcore_axis_name="s")

    @functools.partial(
        pl.kernel,
        mesh=mesh,
        out_type=jax.ShapeDtypeStruct((B, C, NI, NJ, _PX), jnp.float32),
        compiler_params=pltpu.CompilerParams(
            needs_layout_passes=False,
            use_tc_tiling_on_sc=False,
        ),
        scratch_types=[
            pltpu.VMEM((cv, K), jnp.float32),
            pltpu.VMEM_SHARED((B * C, K), jnp.float32),
            pltpu.VMEM((2, _PX), jnp.int32),
            pltpu.VMEM((2, cq_sz, _PX), jnp.float32),
            pltpu.SemaphoreType.DMA,
            pltpu.SemaphoreType.DMA,
            pltpu.SemaphoreType.DMA,
            pltpu.SemaphoreType.DMA,
            pltpu.SemaphoreType.DMA,
        ],
    )
    def k(pooled_hbm, spx_hbm, out_hbm, table_v, tab_sh, idx_v, obuf_v,
          isem0, isem1, osem0, osem1, gsem):
        isems = (isem0, isem1)
        osems = (osem0, osem1)
        sid = lax.axis_index("s")
        wid = sid * _NC + lax.axis_index("c")
        b = wid // (2 * n_cq)
        rem = wid % (2 * n_cq)
        cq = rem // 2
        half = rem % 2
        c0 = cq * cq_sz
        bid0 = half * blocks_per_tile

        @pl.when(sid == 0)
        def _fill_shared():
            pltpu.sync_copy(pooled_hbm, tab_sh)

        pltpu.sync_copy(
            pooled_hbm.at[pl.ds(b * C + c0, cv), :], table_v
        )
        plsc.subcore_barrier()

        pltpu.async_copy(
            spx_hbm.at[b, bid0 // NJ, bid0 % NJ], idx_v.at[0], isems[0]
        )

        def pair_body(p, _):
            for s in (0, 1):
                bid = bid0 + 2 * p + s
                i = bid // NJ
                j = bid % NJ
                pltpu.make_async_copy(
                    spx_hbm.at[b, i, j], idx_v.at[s], isems[s]
                ).wait()

                @pl.when(2 * p + s + 1 < blocks_per_tile)
                def _prefetch():
                    nbid = bid + 1
                    pltpu.async_copy(
                        spx_hbm.at[b, nbid // NJ, nbid % NJ],
                        idx_v.at[1 - s],
                        isems[1 - s],
                    )

                @pl.when(2 * p + s >= 2)
                def _drain():
                    pbid = bid - 2
                    pltpu.make_async_copy(
                        obuf_v.at[s],
                        out_hbm.at[b, pl.ds(c0, cq_sz), pbid // NJ, pbid % NJ],
                        osems[s],
                    ).wait()

                for u in range(_CS):
                    pltpu.async_copy(
                        tab_sh.at[b * C + c0 + cv + u].at[idx_v.at[s]],
                        obuf_v.at[s, cv + u],
                        gsem,
                    )

                pend = []
                for q in range(_PX // 16):
                    iv = idx_v[s, pl.ds(q * 16, 16)]
                    for c in range(cv):
                        v = plsc.load_gather(table_v.at[c], [iv])
                        pend.append((v, c, q))
                        if len(pend) > _G:
                            pv, pc, pq = pend.pop(0)
                            obuf_v[s, pc, pl.ds(pq * 16, 16)] = pv
                for pv, pc, pq in pend:
                    obuf_v[s, pc, pl.ds(pq * 16, 16)] = pv

                for u in range(_CS):
                    pltpu.make_async_copy(
                        tab_sh.at[b * C + c0 + cv + u].at[idx_v.at[s]],
                        obuf_v.at[s, cv + u],
                        gsem,
                    ).wait()

                pltpu.async_copy(
                    obuf_v.at[s],
                    out_hbm.at[b, pl.ds(c0, cq_sz), i, j],
                    osems[s],
                )
            return 0

        lax.fori_loop(0, blocks_per_tile // 2, pair_body, 0)

        for s in (0, 1):
            bid = bid0 + blocks_per_tile - 2 + s
            pltpu.make_async_copy(
                obuf_v.at[s],
                out_hbm.at[b, pl.ds(c0, cq_sz), bid // NJ, bid % NJ],
                osems[s],
            ).wait()

    return k(pooled2, spx5)


def kernel(pooled, spx):
    B, C, K = pooled.shape
    _, H, W = spx.shape
    NI, NJ = H // _TH, W // _TW
    spx5 = (
        spx.reshape(B, NI, _TH, NJ, _TW)
        .transpose(0, 1, 3, 2, 4)
        .reshape(B, NI, NJ, _PX)
    )
    out5 = _suppix_unpool_sc(pooled.reshape(B * C, K), spx5, B, C, K, H, W)
    return (
        out5.reshape(B, C, NI, NJ, _TH, _TW)
        .transpose(0, 1, 2, 4, 3, 5)
        .reshape(B, C, H, W)
    )


# speedup vs baseline: 1.1340x; 1.1340x over previous
import functools

import jax
import jax.numpy as jnp
from jax import lax
from jax.experimental import pallas as pl
from jax.experimental.pallas import tpu as pltpu
from jax.experimental.pallas import tpu_sc as plsc

_NC = 2
_NS = 16
_NW = _NC * _NS

_TH = 8
_TW = 128
_PX = _TH * _TW
_G = 8
_CS = 5


def _suppix_unpool_sc(pooled2, spx5, B, C, K, H, W):
    NI = H // _TH
    NJ = W // _TW
    n_blocks = NI * NJ
    n_cq = _NW // B // 2
    cq_sz = C // n_cq
    cv = cq_sz - _CS
    blocks_per_tile = n_blocks // 2

    mesh = plsc.VectorSubcoreMesh(core_axis_name="c", subcore_axis_name="s")

    @functools.partial(
        pl.kernel,
        mesh=mesh,
        out_type=jax.ShapeDtypeStruct((B, C, NI, NJ, _PX), jnp.float32),
        compiler_params=pltpu.CompilerParams(
            needs_layout_passes=False,
            use_tc_tiling_on_sc=False,
        ),
        scratch_types=[
            pltpu.VMEM((cv, K), jnp.float32),
            pltpu.VMEM_SHARED((B * C, K), jnp.float32),
            pltpu.VMEM((2, _PX), jnp.int32),
            pltpu.VMEM((2, cq_sz, _PX), jnp.float32),
            pltpu.SemaphoreType.DMA,
            pltpu.SemaphoreType.DMA,
            pltpu.SemaphoreType.DMA,
            pltpu.SemaphoreType.DMA,
            pltpu.SemaphoreType.DMA,
        ],
    )
    def k(pooled_hbm, spx_hbm, out_hbm, table_v, tab_sh, idx_v, obuf_v,
          isem0, isem1, osem0, osem1, gsem):
        isems = (isem0, isem1)
        osems = (osem0, osem1)
        sid = lax.axis_index("s")
        wid = sid * _NC + lax.axis_index("c")
        b = wid // (2 * n_cq)
        rem = wid % (2 * n_cq)
        cq = rem // 2
        half = rem % 2
        c0 = cq * cq_sz
        bid0 = half * blocks_per_tile

        @pl.when(sid == 0)
        def _fill_shared():
            pltpu.sync_copy(pooled_hbm, tab_sh)

        pltpu.sync_copy(
            pooled_hbm.at[pl.ds(b * C + c0, cv), :], table_v
        )
        plsc.subcore_barrier()

        pltpu.async_copy(
            spx_hbm.at[b, bid0 // NJ, bid0 % NJ], idx_v.at[0], isems[0]
        )

        def pair_body(p, _):
            for s in (0, 1):
                bid = bid0 + 2 * p + s
                i = bid // NJ
                j = bid % NJ
                pltpu.make_async_copy(
                    spx_hbm.at[b, i, j], idx_v.at[s], isems[s]
                ).wait()

                @pl.when(2 * p + s + 1 < blocks_per_tile)
                def _prefetch():
                    nbid = bid + 1
                    pltpu.async_copy(
                        spx_hbm.at[b, nbid // NJ, nbid % NJ],
                        idx_v.at[1 - s],
                        isems[1 - s],
                    )

                @pl.when(2 * p + s >= 2)
                def _drain():
                    pbid = bid - 2
                    pltpu.make_async_copy(
                        obuf_v.at[s],
                        out_hbm.at[b, pl.ds(c0, cq_sz), pbid // NJ, pbid % NJ],
                        osems[s],
                    ).wait()

                for u in range(_CS):
                    pltpu.async_copy(
                        tab_sh.at[b * C + c0 + cv + u].at[idx_v.at[s]],
                        obuf_v.at[s, cv + u],
                        gsem,
                    )

                pend = []
                for q in range(_PX // 16):
                    iv = idx_v[s, pl.ds(q * 16, 16)]
                    for c in range(cv):
                        v = plsc.load_gather(table_v.at[c], [iv])
                        pend.append((v, c, q))
                        if len(pend) > _G:
                            pv, pc, pq = pend.pop(0)
                            obuf_v[s, pc, pl.ds(pq * 16, 16)] = pv
                for pv, pc, pq in pend:
                    obuf_v[s, pc, pl.ds(pq * 16, 16)] = pv

                for u in range(_CS):
                    pltpu.make_async_copy(
                        tab_sh.at[b * C + c0 + cv + u].at[idx_v.at[s]],
                        obuf_v.at[s, cv + u],
                        gsem,
                    ).wait()

                pltpu.async_copy(
                    obuf_v.at[s],
                    out_hbm.at[b, pl.ds(c0, cq_sz), i, j],
                    osems[s],
                )
            return 0

        lax.fori_loop(0, blocks_per_tile // 2, pair_body, 0)

        for s in (0, 1):
            bid = bid0 + blocks_per_tile - 2 + s
            pltpu.make_async_copy(
                obuf_v.at[s],
                out_hbm.at[b, pl.ds(c0, cq_sz), bid // NJ, bid % NJ],
                osems[s],
            ).wait()

    return k(pooled2, spx5)


def kernel(pooled, spx):
    B, C, K = pooled.shape
    _, H, W = spx.shape
    NI, NJ = H // _TH, W // _TW
    spx5 = (
        spx.reshape(B, NI, _TH, NJ, _TW)
        .transpose(0, 1, 3, 2, 4)
        .reshape(B, NI, NJ, _PX)
    )
    out5 = _suppix_unpool_sc(pooled.reshape(B * C, K), spx5, B, C, K, H, W)
    return (
        out5.reshape(B, C, NI, NJ, _TH, _TW)
        .transpose(0, 1, 2, 4, 3, 5)
        .reshape(B, C, H, W)
    )
